# baseline (speedup 1.0000x reference)
import functools

import jax
import jax.numpy as jnp
from jax import lax
from jax.experimental import pallas as pl
from jax.experimental.pallas import tpu as pltpu
from jax.experimental.pallas import tpu_sc as plsc

B_IMG, H_IMG, W_IMG, DIM = 4, 256, 256, 128
TOTAL = B_IMG * H_IMG * W_IMG
MAPN = 262400
DUMP = TOTAL
DEAD = TOTAL + 8
NPAD = 36864
CW = NPAD // 32
SUB = 128
NSUB = CW // SUB
MW = MAPN // 16
SCW = NPAD // 16
OFFS = [((ki - 1) * 2, (kj - 1) * 2) for ki in range(3) for kj in range(3)]


def _scatter_maps(fx, fm):
    mesh = plsc.VectorSubcoreMesh(core_axis_name="c", subcore_axis_name="s",
                                  num_cores=1)

    def body(fx_hbm, fm_hbm, rmx_hbm, rmm_hbm,
             fill_v, ibuf_v, vbuf_v, fbuf_v, sem):
        w = lax.axis_index("s")

        def fill_loop(i, _):
            fill_v[pl.ds(i * 16, 16)] = jnp.full((16,), -1, jnp.int32)
            return 0
        lax.fori_loop(0, MW // 16, fill_loop, 0)
        pltpu.sync_copy(fill_v, rmx_hbm.at[pl.ds(w * MW, MW)])
        pltpu.sync_copy(fill_v, rmm_hbm.at[pl.ds(w * MW, MW)])
        plsc.subcore_barrier()

        base = w * SCW

        def scat_one(src_hbm, dst_hbm):
            pltpu.sync_copy(src_hbm.at[pl.ds(base, SCW)], fbuf_v)

            def sub(s, _):
                def cp(i, _):
                    off = s * SUB + i * 16
                    ibuf_v[pl.ds(i * 16, 16)] = fbuf_v[pl.ds(off, 16)]
                    vbuf_v[pl.ds(i * 16, 16)] = (
                        base + off + lax.iota(jnp.int32, 16))
                    return 0
                lax.fori_loop(0, SUB // 16, cp, 0)
                pltpu.async_copy(vbuf_v, dst_hbm.at[ibuf_v], sem).wait()
                return 0
            lax.fori_loop(0, SCW // SUB, sub, 0)

        scat_one(fx_hbm, rmx_hbm)
        scat_one(fm_hbm, rmm_hbm)

    f = pl.kernel(
        body,
        out_type=(jax.ShapeDtypeStruct((MAPN,), jnp.int32),
                  jax.ShapeDtypeStruct((MAPN,), jnp.int32)),
        mesh=mesh,
        scratch_types=(
            pltpu.VMEM((MW,), jnp.int32),
            pltpu.VMEM((SUB,), jnp.int32),
            pltpu.VMEM((SUB,), jnp.int32),
            pltpu.VMEM((SCW,), jnp.int32),
            pltpu.SemaphoreType.DMA,
        ),
    )
    return f(fx, fm)


def _gather_neighbors(sx, rmx, rmm, xf_ext, mf_ext):
    mesh = plsc.VectorSubcoreMesh(core_axis_name="c", subcore_axis_name="s")
    nzx = xf_ext.shape[0] - 8
    nzm = mf_ext.shape[0] - 8

    def body(sx_hbm, rmx_hbm, rmm_hbm, xf_hbm, mf_hbm, nbx_hbm, nbm_hbm,
             sx_v, nf_v, jx_v, jm_v, jix_v, jim_v, rx_v, rm_v, sem):
        wid = lax.axis_index("s") * 2 + lax.axis_index("c")
        base = wid * CW
        pltpu.sync_copy(sx_hbm.at[pl.ds(base, CW)], sx_v)

        def sub(s, _):
            for o, (di, dj) in enumerate(OFFS):
                def cmp_nf(i, _):
                    sv = sx_v[pl.ds(s * SUB + i * 16, 16)]
                    bq = lax.shift_right_logical(sv, 16)
                    hh = lax.shift_right_logical(sv, 8) & 255
                    ww = sv & 255
                    nh = hh + di
                    nw = ww + dj
                    inb = (nh >= 0) & (nh < H_IMG) & (nw >= 0) & (nw < W_IMG)
                    nhc = jnp.clip(nh, 0, H_IMG - 1)
                    nwc = jnp.clip(nw, 0, W_IMG - 1)
                    nf = (bq << 16) | (nhc << 8) | nwc
                    nf_v[pl.ds(i * 16, 16)] = jnp.where(inb, nf, DEAD)
                    return 0
                lax.fori_loop(0, SUB // 16, cmp_nf, 0)
                pltpu.async_copy(rmx_hbm.at[nf_v], jx_v, sem).wait()
                pltpu.async_copy(rmm_hbm.at[nf_v], jm_v, sem).wait()

                def cmp_ji(i, _):
                    jx = jx_v[pl.ds(i * 16, 16)]
                    jm = jm_v[pl.ds(i * 16, 16)]
                    jix_v[pl.ds(i * 16, 16)] = jnp.where(jx >= 0, jx, nzx)
                    jim_v[pl.ds(i * 16, 16)] = jnp.where(jm >= 0, jm, nzm)
                    return 0
                lax.fori_loop(0, SUB // 16, cmp_ji, 0)
                pltpu.async_copy(xf_hbm.at[jix_v], rx_v, sem).wait()
                pltpu.sync_copy(rx_v, nbx_hbm.at[o, pl.ds(base + s * SUB, SUB)])
                pltpu.async_copy(mf_hbm.at[jim_v], rm_v, sem).wait()
                pltpu.sync_copy(rm_v, nbm_hbm.at[o, pl.ds(base + s * SUB, SUB)])
            return 0
        lax.fori_loop(0, NSUB, sub, 0)

    f = pl.kernel(
        body,
        out_type=(jax.ShapeDtypeStruct((9, NPAD, DIM), jnp.float32),
                  jax.ShapeDtypeStruct((9, NPAD, DIM), jnp.float32)),
        mesh=mesh,
        scratch_types=(
            pltpu.VMEM((CW,), jnp.int32),
            pltpu.VMEM((SUB,), jnp.int32),
            pltpu.VMEM((SUB,), jnp.int32),
            pltpu.VMEM((SUB,), jnp.int32),
            pltpu.VMEM((SUB,), jnp.int32),
            pltpu.VMEM((SUB,), jnp.int32),
            pltpu.VMEM((SUB, DIM), jnp.float32),
            pltpu.VMEM((SUB, DIM), jnp.float32),
            pltpu.SemaphoreType.DMA,
        ),
    )
    return f(sx, rmx, rmm, xf_ext, mf_ext)


def _matmul_tc(nbx, nbm, w9, ws, b2):
    blk = 256
    grid = (NPAD // blk,)

    def body(nbx_ref, nbm_ref, w9_ref, ws_ref, b_ref, out_ref):
        acc = jnp.zeros((blk, DIM), jnp.float32)
        for o in range(9):
            nb = nbx_ref[o] + nbm_ref[o]
            acc += jnp.dot(nb, w9_ref[o], preferred_element_type=jnp.float32)
        out_ref[...] = (jnp.dot(acc, ws_ref[...],
                                preferred_element_type=jnp.float32)
                        + b_ref[0:1, :])

    return pl.pallas_call(
        body,
        grid=grid,
        in_specs=[
            pl.BlockSpec((9, blk, DIM), lambda i: (0, i, 0)),
            pl.BlockSpec((9, blk, DIM), lambda i: (0, i, 0)),
            pl.BlockSpec((9, DIM, DIM), lambda i: (0, 0, 0)),
            pl.BlockSpec((DIM, DIM), lambda i: (0, 0)),
            pl.BlockSpec((8, DIM), lambda i: (0, 0)),
        ],
        out_specs=pl.BlockSpec((blk, DIM), lambda i: (i, 0)),
        out_shape=jax.ShapeDtypeStruct((NPAD, DIM), jnp.float32),
    )(nbx, nbm, w9, ws, b2)


@jax.jit
def kernel(x_features, x_indices, mem_features, mem_indices, W_agg,
           W_smooth, b_smooth):
    n_x = x_features.shape[0]
    n_m = mem_features.shape[0]
    xi = x_indices.astype(jnp.int32)
    mi = mem_indices.astype(jnp.int32)
    flat_x = (xi[:, 0] * H_IMG + xi[:, 1]) * W_IMG + xi[:, 2]
    flat_m = (mi[:, 0] * H_IMG + mi[:, 1]) * W_IMG + mi[:, 2]
    sorted_x = jnp.sort(flat_x)

    fx = jnp.full((NPAD,), DUMP, jnp.int32).at[:n_x].set(flat_x)
    fm = jnp.full((NPAD,), DUMP, jnp.int32).at[:n_m].set(flat_m)
    sx = jnp.zeros((NPAD,), jnp.int32).at[:n_x].set(sorted_x)

    rmx, rmm = _scatter_maps(fx, fm)

    xf_ext = jnp.concatenate(
        [x_features, jnp.zeros((8, DIM), jnp.float32)], axis=0)
    mf_ext = jnp.concatenate(
        [mem_features, jnp.zeros((8, DIM), jnp.float32)], axis=0)

    nbx, nbm = _gather_neighbors(sx, rmx, rmm, xf_ext, mf_ext)

    w9 = W_agg.reshape(9, DIM, DIM)
    b2 = jnp.broadcast_to(b_smooth[None, :], (8, DIM))
    out = _matmul_tc(nbx, nbm, w9, W_smooth, b2)
    return out[:n_x]


# speedup vs baseline: 1.1037x; 1.1037x over previous
import jax
import jax.numpy as jnp
from jax import lax
from jax.experimental import pallas as pl
from jax.experimental.pallas import tpu as pltpu
from jax.experimental.pallas import tpu_sc as plsc

B_IMG, H_IMG, W_IMG, DIM = 4, 256, 256, 128
TOTAL = B_IMG * H_IMG * W_IMG
MAPN = 262400
DUMP = TOTAL
DEAD = TOTAL + 8
NPAD = 36864
CW = NPAD // 32
SUB = 128
NSUB = CW // SUB
MW = MAPN // 16
SCW = NPAD // 16
NMEM = 35000
ZROW = NPAD + NMEM
OFFS = [((ki - 1) * 2, (kj - 1) * 2) for ki in range(3) for kj in range(3)]
OFF8 = [OFFS[k] for k in (0, 1, 2, 3, 5, 6, 7, 8)]


def _build_maps(sx_s, fm):
    mesh = plsc.VectorSubcoreMesh(core_axis_name="c", subcore_axis_name="s",
                                  num_cores=1)
    ndma = SCW // SUB
    depth = 4

    def body(sx_hbm, fm_hbm, rmc_hbm, rmm_hbm,
             fill_v, ibuf_v, vbuf_v, fbuf_v, sem):
        w = lax.axis_index("s")

        def fill_loop(i, _):
            fill_v[pl.ds(i * 16, 16)] = jnp.full((16,), -1, jnp.int32)
            return 0
        lax.fori_loop(0, MW // 16, fill_loop, 0)
        pltpu.sync_copy(fill_v, rmc_hbm.at[pl.ds(w * MW, MW)])
        pltpu.sync_copy(fill_v, rmm_hbm.at[pl.ds(w * MW, MW)])
        plsc.subcore_barrier()

        base = w * SCW

        def stage(src_slot, j, bias):
            def cp(i, _):
                off = j * SUB + i * 16
                ibuf_v[src_slot, pl.ds(i * 16, 16)] = fbuf_v[pl.ds(off, 16)]
                vbuf_v[src_slot, pl.ds(i * 16, 16)] = (
                    bias + base + off + lax.iota(jnp.int32, 16))
                return 0
            lax.fori_loop(0, SUB // 16, cp, 0)

        def scat_pipelined(dsts, bias):
            descs = [None] * (ndma * len(dsts))
            k = 0
            for j in range(ndma):
                slot = j % depth
                if j >= depth:
                    for q in range(len(dsts)):
                        descs[(j - depth) * len(dsts) + q].wait()
                stage(slot, j, bias)
                for q, d in enumerate(dsts):
                    descs[j * len(dsts) + q] = pltpu.async_copy(
                        vbuf_v.at[slot], d.at[ibuf_v.at[slot]], sem)
            for j in range(max(0, ndma - depth), ndma):
                for q in range(len(dsts)):
                    descs[j * len(dsts) + q].wait()

        pltpu.sync_copy(fm_hbm.at[pl.ds(base, SCW)], fbuf_v)
        scat_pipelined([rmm_hbm], 0)
        scat_pipelined([rmc_hbm], NPAD)
        plsc.subcore_barrier()
        pltpu.sync_copy(sx_hbm.at[pl.ds(base, SCW)], fbuf_v)
        scat_pipelined([rmc_hbm], 0)

    f = pl.kernel(
        body,
        out_type=(jax.ShapeDtypeStruct((MAPN,), jnp.int32),
                  jax.ShapeDtypeStruct((MAPN,), jnp.int32)),
        mesh=mesh,
        scratch_types=(
            pltpu.VMEM((MW,), jnp.int32),
            pltpu.VMEM((depth, SUB), jnp.int32),
            pltpu.VMEM((depth, SUB), jnp.int32),
            pltpu.VMEM((SCW,), jnp.int32),
            pltpu.SemaphoreType.DMA,
        ),
    )
    return f(sx_s, fm)


def _build_xpart(sx_g, perm, rmm, xf_ext, mf_ext):
    mesh = plsc.VectorSubcoreMesh(core_axis_name="c", subcore_axis_name="s")

    def body(sx_hbm, pm_hbm, rmm_hbm, xf_hbm, mf_hbm, xp_hbm,
             sx_v, pm_v, ib_v, ov_v, jm_v, xr_v, mr_v, semi, semr, semw):
        wid = lax.axis_index("s") * 2 + lax.axis_index("c")
        base = wid * CW
        pltpu.sync_copy(sx_hbm.at[pl.ds(base, CW)], sx_v)
        pltpu.sync_copy(pm_hbm.at[pl.ds(base, CW)], pm_v)

        wb = [None, None]
        for s in range(NSUB):
            sl = s % 2

            def cp(i, _):
                ib_v[0, pl.ds(i * 16, 16)] = sx_v[pl.ds(s * SUB + i * 16, 16)]
                ib_v[1, pl.ds(i * 16, 16)] = pm_v[pl.ds(s * SUB + i * 16, 16)]
                return 0
            lax.fori_loop(0, SUB // 16, cp, 0)
            govl = pltpu.async_copy(rmm_hbm.at[ib_v.at[0]], ov_v, semi)
            if wb[sl] is not None:
                wb[sl].wait()
            gx = pltpu.async_copy(xf_hbm.at[ib_v.at[1]], xr_v.at[sl], semr)
            govl.wait()

            def cj(i, _):
                ov = ov_v[pl.ds(i * 16, 16)]
                jm_v[pl.ds(i * 16, 16)] = jnp.where(ov >= 0, ov, NMEM)
                return 0
            lax.fori_loop(0, SUB // 16, cj, 0)
            gm = pltpu.async_copy(mf_hbm.at[jm_v], mr_v, semr)
            gx.wait()
            gm.wait()

            def addrow(i, _):
                def addv(q, _):
                    xr_v[sl, i, pl.ds(q * 16, 16)] = (
                        xr_v[sl, i, pl.ds(q * 16, 16)]
                        + mr_v[i, pl.ds(q * 16, 16)])
                    return 0
                lax.fori_loop(0, DIM // 16, addv, 0)
                return 0
            lax.fori_loop(0, SUB, addrow, 0)
            wb[sl] = pltpu.async_copy(
                xr_v.at[sl], xp_hbm.at[pl.ds(base + s * SUB, SUB)], semw)
        for d in wb:
            if d is not None:
                d.wait()

    f = pl.kernel(
        body,
        out_type=jax.ShapeDtypeStruct((NPAD, DIM), jnp.float32),
        mesh=mesh,
        scratch_types=(
            pltpu.VMEM((CW,), jnp.int32),
            pltpu.VMEM((CW,), jnp.int32),
            pltpu.VMEM((2, SUB), jnp.int32),
            pltpu.VMEM((SUB,), jnp.int32),
            pltpu.VMEM((SUB,), jnp.int32),
            pltpu.VMEM((2, SUB, DIM), jnp.float32),
            pltpu.VMEM((SUB, DIM), jnp.float32),
            pltpu.SemaphoreType.DMA,
            pltpu.SemaphoreType.DMA,
            pltpu.SemaphoreType.DMA,
        ),
    )
    return f(sx_g, perm, rmm, xf_ext, mf_ext)


def _gather_neighbors(sx_g, rmc, uf):
    mesh = plsc.VectorSubcoreMesh(core_axis_name="c", subcore_axis_name="s")
    depth = 3

    def body(sx_hbm, rmc_hbm, uf_hbm, nb_hbm,
             smap, mb_v, sx_v, nf8_v, mv8_v, ji8_v, rows_v, semm, semr, semw):
        sid = lax.axis_index("s")
        pltpu.sync_copy(rmc_hbm.at[pl.ds(sid * MW, MW)], mb_v)
        pltpu.sync_copy(mb_v, smap.at[pl.ds(sid * MW, MW)])
        plsc.subcore_barrier()

        wid = sid * 2 + lax.axis_index("c")
        base = wid * CW
        pltpu.sync_copy(sx_hbm.at[pl.ds(base, CW)], sx_v)

        def sub(s, _):
            def cmp_nf(i, _):
                sv = sx_v[pl.ds(s * SUB + i * 16, 16)]
                bq = lax.shift_right_logical(sv, 16)
                hh = lax.shift_right_logical(sv, 8) & 255
                ww = sv & 255
                for o, (di, dj) in enumerate(OFF8):
                    nh = hh + di
                    nw = ww + dj
                    inb = (nh >= 0) & (nh < H_IMG) & (nw >= 0) & (nw < W_IMG)
                    nhc = jnp.clip(nh, 0, H_IMG - 1)
                    nwc = jnp.clip(nw, 0, W_IMG - 1)
                    nf = (bq << 16) | (nhc << 8) | nwc
                    nf8_v[o, pl.ds(i * 16, 16)] = jnp.where(inb, nf, DEAD)
                return 0
            lax.fori_loop(0, SUB // 16, cmp_nf, 0)

            mg = [pltpu.async_copy(smap.at[nf8_v.at[o]], mv8_v.at[o], semm)
                  for o in range(8)]
            for d in mg:
                d.wait()

            def cmp_ji(i, _):
                for o in range(8):
                    mv = mv8_v[o, pl.ds(i * 16, 16)]
                    ji8_v[o, pl.ds(i * 16, 16)] = jnp.where(mv >= 0, mv, ZROW)
                return 0
            lax.fori_loop(0, SUB // 16, cmp_ji, 0)

            gd = [None] * 8
            wd = [None] * 8
            for o in range(8):
                if o >= depth:
                    wd[o - depth].wait()
                gd[o] = pltpu.async_copy(
                    uf_hbm.at[ji8_v.at[o]], rows_v.at[o % depth], semr)
                if o >= 1:
                    gd[o - 1].wait()
                    wd[o - 1] = pltpu.async_copy(
                        rows_v.at[(o - 1) % depth],
                        nb_hbm.at[o - 1, pl.ds(base + s * SUB, SUB)], semw)
            gd[7].wait()
            wd[7] = pltpu.async_copy(
                rows_v.at[7 % depth],
                nb_hbm.at[7, pl.ds(base + s * SUB, SUB)], semw)
            for o in range(8 - depth, 8):
                wd[o].wait()
            return 0
        lax.fori_loop(0, NSUB, sub, 0)

    f = pl.kernel(
        body,
        out_type=jax.ShapeDtypeStruct((8, NPAD, DIM), jnp.float32),
        mesh=mesh,
        scratch_types=(
            pltpu.VMEM_SHARED((MAPN,), jnp.int32),
            pltpu.VMEM((MW,), jnp.int32),
            pltpu.VMEM((CW,), jnp.int32),
            pltpu.VMEM((8, SUB), jnp.int32),
            pltpu.VMEM((8, SUB), jnp.int32),
            pltpu.VMEM((8, SUB), jnp.int32),
            pltpu.VMEM((depth, SUB, DIM), jnp.float32),
            pltpu.SemaphoreType.DMA,
            pltpu.SemaphoreType.DMA,
            pltpu.SemaphoreType.DMA,
        ),
    )
    return f(sx_g, rmc, uf)


def _matmul_tc(nb, xpart, w8, w4, ws, b2):
    blk = 256
    grid = (NPAD // blk,)

    def body(nb_ref, xp_ref, w8_ref, w4_ref, ws_ref, b_ref, out_ref):
        acc = jnp.dot(xp_ref[...], w4_ref[...],
                      preferred_element_type=jnp.float32)
        for o in range(8):
            acc += jnp.dot(nb_ref[o], w8_ref[o],
                           preferred_element_type=jnp.float32)
        out_ref[...] = (jnp.dot(acc, ws_ref[...],
                                preferred_element_type=jnp.float32)
                        + b_ref[0:1, :])

    return pl.pallas_call(
        body,
        grid=grid,
        in_specs=[
            pl.BlockSpec((8, blk, DIM), lambda i: (0, i, 0)),
            pl.BlockSpec((blk, DIM), lambda i: (i, 0)),
            pl.BlockSpec((8, DIM, DIM), lambda i: (0, 0, 0)),
            pl.BlockSpec((DIM, DIM), lambda i: (0, 0)),
            pl.BlockSpec((DIM, DIM), lambda i: (0, 0)),
            pl.BlockSpec((8, DIM), lambda i: (0, 0)),
        ],
        out_specs=pl.BlockSpec((blk, DIM), lambda i: (i, 0)),
        out_shape=jax.ShapeDtypeStruct((NPAD, DIM), jnp.float32),
    )(nb, xpart, w8, w4, ws, b2)


@jax.jit
def kernel(x_features, x_indices, mem_features, mem_indices, W_agg,
           W_smooth, b_smooth):
    n_x = x_features.shape[0]
    xi = x_indices.astype(jnp.int32)
    mi = mem_indices.astype(jnp.int32)
    flat_x = (xi[:, 0] * H_IMG + xi[:, 1]) * W_IMG + xi[:, 2]
    flat_m = (mi[:, 0] * H_IMG + mi[:, 1]) * W_IMG + mi[:, 2]
    perm = jnp.argsort(flat_x)
    sorted_x = flat_x[perm]

    sx_s = jnp.full((NPAD,), DUMP, jnp.int32).at[:n_x].set(sorted_x)
    sx_g = jnp.zeros((NPAD,), jnp.int32).at[:n_x].set(sorted_x)
    fm = jnp.full((NPAD,), DUMP, jnp.int32).at[:NMEM].set(flat_m)
    pm = jnp.full((NPAD,), n_x, jnp.int32).at[:n_x].set(
        perm.astype(jnp.int32))

    rmc, rmm = _build_maps(sx_s, fm)

    xf_ext = jnp.concatenate(
        [x_features, jnp.zeros((8, DIM), jnp.float32)], axis=0)
    mf_ext = jnp.concatenate(
        [mem_features, jnp.zeros((8, DIM), jnp.float32)], axis=0)

    xpart = _build_xpart(sx_g, pm, rmm, xf_ext, mf_ext)

    uf = jnp.concatenate(
        [xpart, mem_features, jnp.zeros((8, DIM), jnp.float32)], axis=0)

    nb = _gather_neighbors(sx_g, rmc, uf)

    w9 = W_agg.reshape(9, DIM, DIM)
    w8 = w9[jnp.array([0, 1, 2, 3, 5, 6, 7, 8])]
    w4 = w9[4]
    b2 = jnp.broadcast_to(b_smooth[None, :], (8, DIM))
    out = _matmul_tc(nb, xpart, w8, w4, W_smooth, b2)
    return out[:n_x]
